# baseline (speedup 1.0000x reference)
import jax
import jax.numpy as jnp
from jax.experimental import pallas as pl
from jax.experimental.pallas import tpu as pltpu


def _cdiv(a, b):
    return -(-a // b)


def _subsample_kernel(*refs):
    o_ref = refs[-1]
    wo = o_ref.shape[1]
    ctile = refs[0].shape[-1]
    for j, x_ref in enumerate(refs[:-1]):
        o_ref[:, :, j * ctile:(j + 1) * ctile] = (
            x_ref[:, 0, pl.ds(0, wo, stride=2), :])


def kernel(x):
    n, c, h, w = x.shape
    ho, wo = h // 2, w // 2

    xt = jnp.transpose(x, (0, 2, 3, 1)).reshape(n * ho, 2, w, c)

    bp = min(n * ho, 256)
    ctile = 128
    grid = (_cdiv(n * ho, bp),)

    out = pl.pallas_call(
        _subsample_kernel,
        out_shape=jax.ShapeDtypeStruct((n * ho, wo, c), x.dtype),
        grid=grid,
        in_specs=[pl.BlockSpec((bp, 1, w, ctile),
                               (lambda i, j=j: (i, 0, 0, j)))
                  for j in range(c // ctile)],
        out_specs=pl.BlockSpec((bp, wo, c), lambda i: (i, 0, 0)),
        compiler_params=pltpu.CompilerParams(
            dimension_semantics=("parallel",)),
        cost_estimate=pl.CostEstimate(
            flops=0, transcendentals=0,
            bytes_accessed=(n * ho * w * c + n * ho * wo * c) * x.dtype.itemsize),
    )(*([xt] * (c // ctile)))
    return [jnp.transpose(out.reshape(n, ho, wo, c), (0, 3, 1, 2))]


# speedup vs baseline: 1.0363x; 1.0363x over previous
import jax
import jax.numpy as jnp
from jax.experimental import pallas as pl
from jax.experimental.pallas import tpu as pltpu


def _cdiv(a, b):
    return -(-a // b)


def _subsample_kernel(x_ref, o_ref):
    wo = o_ref.shape[1]
    o_ref[...] = x_ref[:, 0, pl.ds(0, wo, stride=2), :]


def kernel(x):
    n, c, h, w = x.shape
    ho, wo = h // 2, w // 2

    xt = jnp.transpose(x, (0, 2, 3, 1)).reshape(n * ho, 2, w, c)

    bp = min(n * ho, 256)
    ctile = 128
    grid = (_cdiv(n * ho, bp), c // ctile)

    out = pl.pallas_call(
        _subsample_kernel,
        out_shape=jax.ShapeDtypeStruct((n * ho, wo, c), x.dtype),
        grid=grid,
        in_specs=[pl.BlockSpec((bp, 1, w, ctile),
                               lambda i, j: (i, 0, 0, j))],
        out_specs=pl.BlockSpec((bp, wo, ctile), lambda i, j: (i, 0, j)),
        compiler_params=pltpu.CompilerParams(
            dimension_semantics=("parallel", "parallel")),
        cost_estimate=pl.CostEstimate(
            flops=0, transcendentals=0,
            bytes_accessed=(n * ho * w * c + n * ho * wo * c) * x.dtype.itemsize),
    )(xt)
    return [jnp.transpose(out.reshape(n, ho, wo, c), (0, 3, 1, 2))]
